# baseline (speedup 1.0000x reference)
import jax
import jax.numpy as jnp
from jax.experimental import pallas as pl
from jax.experimental.pallas import tpu as pltpu


_NCHUNKS = 32


def _copy_body(in_ref, out_ref, sems):
    copies = [
        pltpu.make_async_copy(in_ref.at[i], out_ref.at[i], sems.at[i])
        for i in range(_NCHUNKS)
    ]
    for c in copies:
        c.start()
    for c in copies:
        c.wait()


def kernel(sequence):
    b, t, d = sequence.shape
    rows = b * t
    chunked = sequence.reshape(_NCHUNKS, rows // _NCHUNKS, d)
    out = pl.pallas_call(
        _copy_body,
        out_shape=jax.ShapeDtypeStruct(chunked.shape, chunked.dtype),
        in_specs=[pl.BlockSpec(memory_space=pl.ANY)],
        out_specs=pl.BlockSpec(memory_space=pl.ANY),
        scratch_shapes=[pltpu.SemaphoreType.DMA((_NCHUNKS,))],
    )(chunked)
    return out.reshape(b, t, d)


# speedup vs baseline: 47.0849x; 47.0849x over previous
import jax
import jax.numpy as jnp
from jax.experimental import pallas as pl
from jax.experimental.pallas import tpu as pltpu


_BLOCK_ROWS = 1024


def _copy_body(in_ref, out_ref):
    out_ref[...] = in_ref[...]


def kernel(sequence):
    b, t, d = sequence.shape
    rows = b * t
    flat = sequence.reshape(rows, d)
    grid = (rows // _BLOCK_ROWS,)
    out = pl.pallas_call(
        _copy_body,
        out_shape=jax.ShapeDtypeStruct(flat.shape, flat.dtype),
        grid=grid,
        in_specs=[pl.BlockSpec((_BLOCK_ROWS, d), lambda i: (i, 0))],
        out_specs=pl.BlockSpec((_BLOCK_ROWS, d), lambda i: (i, 0)),
    )(flat)
    return out.reshape(b, t, d)
